# Initial kernel scaffold; baseline (speedup 1.0000x reference)
import jax
import jax.numpy as jnp
from jax.experimental import pallas as pl


def kernel(gram, context, word_table, context_table, word_bias, context_bias):
    raise NotImplementedError("write your pallas kernel here")


# speedup vs baseline: 9.8344x; 9.8344x over previous
import functools

import jax
import jax.numpy as jnp
from jax import lax
from jax.experimental import pallas as pl
from jax.experimental.pallas import tpu as pltpu
from jax.experimental.pallas import tpu_sc as plsc

S = 16384
Y = 20
D = 64
NW = 32
R = S // NW
G = 32
NCHUNK = R // G
PAIRS = G * Y
NSTEP = PAIRS // 128


def _sc_body(gram_h, ctx_h, wt_h, ct_h, wb_h, cb_h, out_h,
             widx_v, wrows_v, cidx_v, crows_v, wb_v, cb_v, dots_v, sem):
  wid = lax.axis_index("s") * 2 + lax.axis_index("c")
  wbase = wid * R

  lanes = lax.iota(jnp.int32, 16)
  lane15 = lanes == 15

  def chunk_body(ci, carry):
    gbase = pl.multiple_of(wbase + ci * G, G)
    pbase = pl.multiple_of(gbase * Y, 128)

    pltpu.sync_copy(gram_h.at[pl.ds(gbase, G)], widx_v)
    for j in range(NSTEP):
      pltpu.sync_copy(ctx_h.at[pl.ds(pbase + j * 128, 128)], cidx_v.at[j])

    cps = []
    cps.append(pltpu.async_copy(wt_h.at[widx_v], wrows_v, sem))
    cps.append(pltpu.async_copy(wb_h.at[widx_v], wb_v, sem))
    for j in range(NSTEP):
      cps.append(pltpu.async_copy(
          ct_h.at[cidx_v.at[j]], crows_v.at[pl.ds(j * 128, 128)], sem))
      cps.append(pltpu.async_copy(
          cb_h.at[cidx_v.at[j]], cb_v.at[pl.ds(j * 128, 128)], sem))
    for cp in cps:
      cp.wait()

    def g_body(g, c):
      w0 = wrows_v[g, pl.ds(0, 16)]
      w1 = wrows_v[g, pl.ds(16, 16)]
      w2 = wrows_v[g, pl.ds(32, 16)]
      w3 = wrows_v[g, pl.ds(48, 16)]
      p0 = g * Y
      for y in range(Y):
        p = p0 + y
        acc = w0 * crows_v[p, pl.ds(0, 16)]
        acc = acc + w1 * crows_v[p, pl.ds(16, 16)]
        acc = acc + w2 * crows_v[p, pl.ds(32, 16)]
        acc = acc + w3 * crows_v[p, pl.ds(48, 16)]
        cs = plsc.cumsum(acc)
        plsc.store_scatter(dots_v, [jnp.full((16,), p, jnp.int32)], cs,
                           mask=lane15)
      return c
    lax.fori_loop(0, G, g_body, 0, unroll=False)

    for t in range(PAIRS // 16):
      flat = t * 16 + lanes
      gidx = flat // Y
      wbx = plsc.load_gather(wb_v, [gidx])
      dots_v[pl.ds(t * 16, 16)] = (
          dots_v[pl.ds(t * 16, 16)] + wbx + cb_v[pl.ds(t * 16, 16)])

    pltpu.sync_copy(dots_v, out_h.at[pl.ds(pbase, PAIRS)])
    return carry

  lax.fori_loop(0, NCHUNK, chunk_body, 0, unroll=False)


@jax.jit
def _sc_call(gram_flat, ctx_flat, word_table, context_table, word_bias,
             context_bias):
  mesh = plsc.VectorSubcoreMesh(core_axis_name="c", subcore_axis_name="s")
  return pl.kernel(
      _sc_body,
      out_type=jax.ShapeDtypeStruct((S * Y,), jnp.float32),
      mesh=mesh,
      compiler_params=pltpu.CompilerParams(
          needs_layout_passes=False, use_tc_tiling_on_sc=False),
      scratch_types=[
          pltpu.VMEM((G,), jnp.int32),
          pltpu.VMEM((G, D), jnp.float32),
          pltpu.VMEM((NSTEP, 128), jnp.int32),
          pltpu.VMEM((PAIRS, D), jnp.float32),
          pltpu.VMEM((G,), jnp.float32),
          pltpu.VMEM((PAIRS,), jnp.float32),
          pltpu.VMEM((PAIRS,), jnp.float32),
          pltpu.SemaphoreType.DMA,
      ],
  )(gram_flat, ctx_flat, word_table, context_table, word_bias, context_bias)


def kernel(gram, context, word_table, context_table, word_bias, context_bias):
  gram_flat = gram.reshape(S).astype(jnp.int32)
  ctx_flat = context.reshape(S * Y).astype(jnp.int32)
  out_flat = _sc_call(gram_flat, ctx_flat, word_table, context_table,
                      word_bias, context_bias)
  return out_flat.reshape(S, Y)


# speedup vs baseline: 11.0795x; 1.1266x over previous
import jax
import jax.numpy as jnp
from jax import lax
from jax.experimental import pallas as pl
from jax.experimental.pallas import tpu as pltpu
from jax.experimental.pallas import tpu_sc as plsc

S = 16384
Y = 20
D = 64
NW = 32
R = S // NW
G = 32
NCHUNK = R // G
PAIRS = G * Y
NSTEP = PAIRS // 128


def _sc_body(gram_h, ctx_h, wt_h, ct_h, wb_h, cb_h, out_h,
             widx0, wrows0, cidx0, crows0, wbv0, cbv0, dots0,
             widx1, wrows1, cidx1, crows1, wbv1, cbv1, dots1,
             sem0, sem1):
  wid = lax.axis_index("s") * 2 + lax.axis_index("c")
  wbase = wid * R

  lanes = lax.iota(jnp.int32, 16)
  lane15 = lanes == 15
  buf0 = (widx0, wrows0, cidx0, crows0, wbv0, cbv0, dots0, sem0)
  buf1 = (widx1, wrows1, cidx1, crows1, wbv1, cbv1, dots1, sem1)

  def pbase_of(ci):
    gbase = pl.multiple_of(wbase + ci * G, G)
    return pl.multiple_of(gbase * Y, 128)

  def load_idx_and_fire(ci, buf):
    widx_v, wrows_v, cidx_v, crows_v, wb_v, cb_v, _, sem = buf
    gbase = pl.multiple_of(wbase + ci * G, G)
    pbase = pbase_of(ci)
    pltpu.sync_copy(gram_h.at[pl.ds(gbase, G)], widx_v)
    for j in range(NSTEP):
      pltpu.sync_copy(ctx_h.at[pl.ds(pbase + j * 128, 128)], cidx_v.at[j])
    pltpu.async_copy(wt_h.at[widx_v], wrows_v, sem)
    pltpu.async_copy(wb_h.at[widx_v], wb_v, sem)
    for j in range(NSTEP):
      pltpu.async_copy(ct_h.at[cidx_v.at[j]],
                       crows_v.at[pl.ds(j * 128, 128)], sem)
      pltpu.async_copy(cb_h.at[cidx_v.at[j]],
                       cb_v.at[pl.ds(j * 128, 128)], sem)

  def drain(buf):
    widx_v, wrows_v, cidx_v, crows_v, wb_v, cb_v, _, sem = buf
    pltpu.make_async_copy(wt_h.at[pl.ds(0, G)], wrows_v, sem).wait()
    pltpu.make_async_copy(wb_h.at[pl.ds(0, G)], wb_v, sem).wait()
    for j in range(NSTEP):
      pltpu.make_async_copy(ct_h.at[pl.ds(0, 128)],
                            crows_v.at[pl.ds(j * 128, 128)], sem).wait()
      pltpu.make_async_copy(cb_h.at[pl.ds(0, 128)],
                            cb_v.at[pl.ds(j * 128, 128)], sem).wait()

  def compute(ci, buf):
    widx_v, wrows_v, cidx_v, crows_v, wb_v, cb_v, dots_v, _ = buf
    pbase = pbase_of(ci)

    def g_body(g, c):
      w0 = wrows_v[g, pl.ds(0, 16)]
      w1 = wrows_v[g, pl.ds(16, 16)]
      w2 = wrows_v[g, pl.ds(32, 16)]
      w3 = wrows_v[g, pl.ds(48, 16)]
      p0 = g * Y
      for y in range(Y):
        p = p0 + y
        acc = w0 * crows_v[p, pl.ds(0, 16)]
        acc = acc + w1 * crows_v[p, pl.ds(16, 16)]
        acc = acc + w2 * crows_v[p, pl.ds(32, 16)]
        acc = acc + w3 * crows_v[p, pl.ds(48, 16)]
        cs = plsc.cumsum(acc)
        plsc.store_scatter(dots_v, [jnp.full((16,), p, jnp.int32)], cs,
                           mask=lane15)
      return c
    lax.fori_loop(0, G, g_body, 0, unroll=False)

    for t in range(PAIRS // 16):
      flat = t * 16 + lanes
      gidx = flat // Y
      wbx = plsc.load_gather(wb_v, [gidx])
      dots_v[pl.ds(t * 16, 16)] = (
          dots_v[pl.ds(t * 16, 16)] + wbx + cb_v[pl.ds(t * 16, 16)])

    pltpu.sync_copy(dots_v, out_h.at[pl.ds(pbase, PAIRS)])

  load_idx_and_fire(0, buf0)

  def loop_body(i, carry):
    a = 2 * i
    load_idx_and_fire(a + 1, buf1)
    drain(buf0)
    compute(a, buf0)

    @pl.when(i < NCHUNK // 2 - 1)
    def _():
      load_idx_and_fire(a + 2, buf0)

    drain(buf1)
    compute(a + 1, buf1)
    return carry

  lax.fori_loop(0, NCHUNK // 2, loop_body, 0, unroll=False)


@jax.jit
def _sc_call(gram_flat, ctx_flat, word_table, context_table, word_bias,
             context_bias):
  mesh = plsc.VectorSubcoreMesh(core_axis_name="c", subcore_axis_name="s")
  dbuf = [
      pltpu.VMEM((G,), jnp.int32),
      pltpu.VMEM((G, D), jnp.float32),
      pltpu.VMEM((NSTEP, 128), jnp.int32),
      pltpu.VMEM((PAIRS, D), jnp.float32),
      pltpu.VMEM((G,), jnp.float32),
      pltpu.VMEM((PAIRS,), jnp.float32),
      pltpu.VMEM((PAIRS,), jnp.float32),
  ]
  return pl.kernel(
      _sc_body,
      out_type=jax.ShapeDtypeStruct((S * Y,), jnp.float32),
      mesh=mesh,
      compiler_params=pltpu.CompilerParams(
          needs_layout_passes=False, use_tc_tiling_on_sc=False),
      scratch_types=dbuf + dbuf + [
          pltpu.SemaphoreType.DMA,
          pltpu.SemaphoreType.DMA,
      ],
  )(gram_flat, ctx_flat, word_table, context_table, word_bias, context_bias)


def kernel(gram, context, word_table, context_table, word_bias, context_bias):
  gram_flat = gram.reshape(S).astype(jnp.int32)
  ctx_flat = context.reshape(S * Y).astype(jnp.int32)
  out_flat = _sc_call(gram_flat, ctx_flat, word_table, context_table,
                      word_bias, context_bias)
  return out_flat.reshape(S, Y)
